# baseline (speedup 1.0000x reference)
import functools
import math

import jax
import jax.numpy as jnp
from jax import lax
from jax.experimental import pallas as pl
from jax.experimental.pallas import tpu as pltpu
from jax.experimental.pallas import tpu_sc as plsc

D_MODEL = 768
SEQ_LEN = 2048
SCALE = math.sqrt(D_MODEL)

NUM_CORES = 2
NUM_SUBCORES = 16
NUM_WORKERS = NUM_CORES * NUM_SUBCORES
LANES = 16

B_TOTAL = 4 * SEQ_LEN
PER_WORKER = B_TOTAL // NUM_WORKERS
CHUNK = 64
N_CHUNKS = PER_WORKER // CHUNK


def _build_lookup():
    mesh = plsc.VectorSubcoreMesh(core_axis_name="c", subcore_axis_name="s")

    @functools.partial(
        pl.kernel,
        out_type=jax.ShapeDtypeStruct((B_TOTAL, D_MODEL), jnp.float32),
        mesh=mesh,
        scratch_types=[
            pltpu.VMEM((PER_WORKER,), jnp.int32),
            pltpu.VMEM((CHUNK, D_MODEL), jnp.float32),
            pltpu.VMEM((CHUNK, D_MODEL), jnp.float32),
            pltpu.SemaphoreType.DMA,
        ],
    )
    def emb(ids_hbm, table_hbm, pos_hbm, out_hbm, idx_v, rows_v, pos_v, sem):
        wid = lax.axis_index("s") * NUM_CORES + lax.axis_index("c")
        base = pl.multiple_of(wid * PER_WORKER, PER_WORKER)
        pltpu.sync_copy(ids_hbm.at[pl.ds(base, PER_WORKER)], idx_v)
        for c in range(N_CHUNKS):
            row0 = pl.multiple_of(base + c * CHUNK, CHUNK)
            pos0 = pl.multiple_of(lax.rem(row0, SEQ_LEN), CHUNK)
            gat = pltpu.async_copy(
                table_hbm.at[idx_v.at[pl.ds(c * CHUNK, CHUNK)]], rows_v, sem
            )
            pltpu.sync_copy(pos_hbm.at[pl.ds(pos0, CHUNK)], pos_v)
            gat.wait()

            @pl.loop(0, CHUNK)
            def _(r):
                @pl.loop(0, D_MODEL, step=LANES)
                def _(j):
                    rows_v.at[pl.ds(r, 1), pl.ds(j, LANES)][...] = (
                        rows_v.at[pl.ds(r, 1), pl.ds(j, LANES)][...] * SCALE
                        + pos_v.at[pl.ds(r, 1), pl.ds(j, LANES)][...]
                    )

            pltpu.sync_copy(rows_v, out_hbm.at[pl.ds(row0, CHUNK)])

    return emb


_lookup = _build_lookup()


@jax.jit
def kernel(input_ids, token_table, pos_table):
    batch, seq_len = input_ids.shape
    flat_ids = input_ids.reshape(-1).astype(jnp.int32)
    out = _lookup(flat_ids, token_table, pos_table)
    return out.reshape(batch, seq_len, D_MODEL)


# speedup vs baseline: 2.1991x; 2.1991x over previous
import functools
import math

import jax
import jax.numpy as jnp
from jax import lax
from jax.experimental import pallas as pl
from jax.experimental.pallas import tpu as pltpu
from jax.experimental.pallas import tpu_sc as plsc

D_MODEL = 768
SEQ_LEN = 2048
SCALE = math.sqrt(D_MODEL)

NUM_CORES = 2
NUM_SUBCORES = 16
NUM_WORKERS = NUM_CORES * NUM_SUBCORES
LANES = 16

B_TOTAL = 4 * SEQ_LEN
PER_WORKER = B_TOTAL // NUM_WORKERS
CHUNK = 32
N_CHUNKS = PER_WORKER // CHUNK
N_ROWBUF = 3
N_POSBUF = 2


def _build_lookup():
    mesh = plsc.VectorSubcoreMesh(core_axis_name="c", subcore_axis_name="s")

    @functools.partial(
        pl.kernel,
        out_type=jax.ShapeDtypeStruct((B_TOTAL, D_MODEL), jnp.float32),
        mesh=mesh,
        scratch_types=[
            pltpu.VMEM((PER_WORKER,), jnp.int32),
            [pltpu.VMEM((CHUNK, D_MODEL), jnp.float32) for _ in range(N_ROWBUF)],
            [pltpu.VMEM((CHUNK, D_MODEL), jnp.float32) for _ in range(N_POSBUF)],
            [pltpu.SemaphoreType.DMA for _ in range(N_ROWBUF)],
            [pltpu.SemaphoreType.DMA for _ in range(N_POSBUF)],
            [pltpu.SemaphoreType.DMA for _ in range(N_ROWBUF)],
        ],
    )
    def emb(ids_hbm, table_hbm, pos_hbm, out_hbm, idx_v, rows, posb, gsem,
            psem, wsem):
        wid = lax.axis_index("s") * NUM_CORES + lax.axis_index("c")
        base = pl.multiple_of(wid * PER_WORKER, PER_WORKER)
        pltpu.sync_copy(ids_hbm.at[pl.ds(base, PER_WORKER)], idx_v)

        def issue_gather(c):
            b = c % N_ROWBUF
            return pltpu.async_copy(
                table_hbm.at[idx_v.at[pl.ds(c * CHUNK, CHUNK)]], rows[b],
                gsem[b])

        def issue_pos(c):
            b = c % N_POSBUF
            pos0 = pl.multiple_of(
                lax.rem(base + c * CHUNK, SEQ_LEN), CHUNK)
            return pltpu.async_copy(
                pos_hbm.at[pl.ds(pos0, CHUNK)], posb[b], psem[b])

        gh = {0: issue_gather(0), 1: issue_gather(1)}
        ph = {0: issue_pos(0), 1: issue_pos(1)}
        wh = {}
        for c in range(N_CHUNKS):
            b = c % N_ROWBUF
            pb = c % N_POSBUF
            gh[c].wait()
            ph[c].wait()

            @pl.loop(0, CHUNK)
            def _(r):
                for j in range(0, D_MODEL, LANES):
                    rows[b].at[pl.ds(r, 1), pl.ds(j, LANES)][...] = (
                        rows[b].at[pl.ds(r, 1), pl.ds(j, LANES)][...] * SCALE
                        + posb[pb].at[pl.ds(r, 1), pl.ds(j, LANES)][...]
                    )

            row0 = pl.multiple_of(base + c * CHUNK, CHUNK)
            wh[c] = pltpu.async_copy(rows[b], out_hbm.at[pl.ds(row0, CHUNK)],
                                     wsem[b])
            if c + 2 < N_CHUNKS:
                if c >= 1:
                    wh[c - 1].wait()
                gh[c + 2] = issue_gather(c + 2)
                ph[c + 2] = issue_pos(c + 2)

        for c in range(N_CHUNKS - N_ROWBUF, N_CHUNKS):
            wh[c].wait()

    return emb


_lookup = _build_lookup()


@jax.jit
def kernel(input_ids, token_table, pos_table):
    batch, seq_len = input_ids.shape
    flat_ids = input_ids.reshape(-1).astype(jnp.int32)
    out = _lookup(flat_ids, token_table, pos_table)
    return out.reshape(batch, seq_len, D_MODEL)
